# baseline (speedup 1.0000x reference)
import functools
import jax
import jax.numpy as jnp
from jax import lax
from jax.experimental import pallas as pl
from jax.experimental.pallas import tpu as pltpu, tpu_sc as plsc

L = 16
R = 8
F = 1024
NB = 4
NW = 32
NBUF = 3
G = 16


def _sc_body(x_hbm, pe_hbm, out_hbm, pe_v, x_v,
             sp0, sp1, si0, si1, si2, so0, so1, so2):
    sps = (sp0, sp1)
    sis = (si0, si1, si2)
    sos = (so0, so1, so2)

    S = x_hbm.shape[1]
    rows_per_worker = S // NW
    nstep = rows_per_worker // R

    nc = 2
    wid = lax.axis_index("s") * nc + lax.axis_index("c")
    seq0 = wid * rows_per_worker

    def start_in(i):
        k = i % NBUF
        return [
            pltpu.async_copy(x_hbm.at[b, pl.ds(seq0 + i * R, R)],
                             x_v.at[pl.ds((k * NB + b) * R, R)], sis[k])
            for b in range(NB)
        ]

    pe_cp = [None, None]
    pe_cp[0] = pltpu.async_copy(pe_hbm.at[pl.ds(seq0, R)],
                                pe_v.at[pl.ds(0, R)], sps[0])
    in_cp = [None] * nstep
    out_cp = [None] * nstep
    in_cp[0] = start_in(0)
    in_cp[1] = start_in(1)

    for i in range(nstep):
        k = i % NBUF
        p = i % 2
        pe_cp[p].wait()
        if i + 1 < nstep:
            pe_cp[1 - p] = pltpu.async_copy(
                pe_hbm.at[pl.ds(seq0 + (i + 1) * R, R)],
                pe_v.at[pl.ds((1 - p) * R, R)], sps[1 - p])
        for cp in in_cp[i]:
            cp.wait()
        if i + 2 < nstep:
            if i - 1 >= 0:
                for cp in out_cp[i - 1]:
                    cp.wait()
            in_cp[i + 2] = start_in(i + 2)

        ngroups = F // (L * G)
        pe_row0 = p * R
        x_row0 = k * NB * R

        def row_body(it, pe_row0=pe_row0, x_row0=x_row0):
            pr = lax.shift_right_logical(it, 2)
            g = lax.bitwise_and(it, ngroups - 1)
            col0 = lax.mul(g, G * L)
            vecs = [pe_v[pe_row0 + pr, pl.ds(col0 + t * L, L)]
                    for t in range(G)]
            for b in range(NB):
                row = x_row0 + b * R + pr
                for t in range(G):
                    plsc.addupdate(x_v.at[row, pl.ds(col0 + t * L, L)],
                                   vecs[t])

        plsc.parallel_loop(0, R * ngroups, 1, unroll=2)(row_body)

        out_cp[i] = [
            pltpu.async_copy(x_v.at[pl.ds((k * NB + b) * R, R)],
                             out_hbm.at[b, pl.ds(seq0 + i * R, R)], sos[k])
            for b in range(NB)
        ]

    for i in range(nstep - NBUF, nstep):
        if out_cp[i] is not None:
            for cp in out_cp[i]:
                cp.wait()


def kernel(x, pe_table):
    B, S, Feat = x.shape
    mesh = plsc.VectorSubcoreMesh(core_axis_name="c", subcore_axis_name="s")
    k = functools.partial(
        pl.kernel,
        mesh=mesh,
        out_type=jax.ShapeDtypeStruct((B, S, Feat), x.dtype),
        scratch_types=[
            pltpu.VMEM((2 * R, Feat), jnp.float32),
            pltpu.VMEM((NBUF * NB * R, Feat), jnp.float32),
            pltpu.SemaphoreType.DMA,
            pltpu.SemaphoreType.DMA,
            pltpu.SemaphoreType.DMA,
            pltpu.SemaphoreType.DMA,
            pltpu.SemaphoreType.DMA,
            pltpu.SemaphoreType.DMA,
            pltpu.SemaphoreType.DMA,
            pltpu.SemaphoreType.DMA,
        ],
    )(_sc_body)
    return k(x, pe_table)
